# baseline (speedup 1.0000x reference)
import functools

import jax
import jax.numpy as jnp
from jax import lax
from jax.experimental import pallas as pl
from jax.experimental.pallas import tpu as pltpu
from jax.experimental.pallas import tpu_sc as plsc


def _make_sc_kernel(B, D, b_per_w, num_cores):
    mesh = plsc.VectorSubcoreMesh(core_axis_name="c", subcore_axis_name="s")

    @functools.partial(
        pl.kernel,
        out_type=jax.ShapeDtypeStruct((B,), jnp.float32),
        mesh=mesh,
        compiler_params=pltpu.CompilerParams(
            needs_layout_passes=False, use_tc_tiling_on_sc=False
        ),
        scratch_types=[
            pltpu.VMEM((b_per_w,), jnp.int32),
            pltpu.VMEM((b_per_w,), jnp.int32),
            pltpu.VMEM((D, b_per_w), jnp.float32),
            pltpu.VMEM((D, b_per_w), jnp.float32),
            pltpu.VMEM((b_per_w,), jnp.float32),
            pltpu.VMEM((b_per_w,), jnp.float32),
            pltpu.SemaphoreType.DMA,
            pltpu.SemaphoreType.DMA,
            pltpu.SemaphoreType.DMA,
        ],
    )
    def sc_kernel(tableT_hbm, bias_hbm, ida_hbm, idb_hbm, out_hbm,
                  idx_a, idx_b, cols_a, cols_b, bias_v, out_v,
                  sem_a, sem_b, sem_c):
        wid = lax.axis_index("s") * num_cores + lax.axis_index("c")
        base = wid * b_per_w
        pltpu.sync_copy(ida_hbm.at[pl.ds(base, b_per_w)], idx_a)
        pltpu.sync_copy(idb_hbm.at[pl.ds(base, b_per_w)], idx_b)
        bias_copy = pltpu.async_copy(bias_hbm.at[idx_b], bias_v, sem_c)

        copies = []
        for d in range(D):
            copies.append(
                pltpu.async_copy(tableT_hbm.at[d].at[idx_a], cols_a.at[d], sem_a)
            )
            copies.append(
                pltpu.async_copy(tableT_hbm.at[d].at[idx_b], cols_b.at[d], sem_b)
            )
        for c in copies:
            c.wait()
        bias_copy.wait()

        def body(blk, _):
            row0 = blk * 16
            acc = bias_v[pl.ds(row0, 16)]
            for d in range(D):
                acc = acc + cols_a[d, pl.ds(row0, 16)] * cols_b[d, pl.ds(row0, 16)]
            out_v[pl.ds(row0, 16)] = acc
            return _

        lax.fori_loop(0, b_per_w // 16, body, None)
        pltpu.sync_copy(out_v, out_hbm.at[pl.ds(base, b_per_w)])

    return sc_kernel


def kernel(embedding_matrix, bias, node_id, node_neighbor_id):
    B = node_id.shape[0]
    D = embedding_matrix.shape[1]
    info = plsc.get_sparse_core_info()
    nw = info.num_cores * info.num_subcores
    b_per_w = B // nw
    sc_kernel = _make_sc_kernel(B, D, b_per_w, info.num_cores)
    return sc_kernel(
        embedding_matrix.T,
        bias,
        node_id.astype(jnp.int32),
        node_neighbor_id.astype(jnp.int32),
    )


# speedup vs baseline: 3.1063x; 3.1063x over previous
import functools

import jax
import jax.numpy as jnp
from jax import lax
from jax.experimental import pallas as pl
from jax.experimental.pallas import tpu as pltpu
from jax.experimental.pallas import tpu_sc as plsc

_OSH = 15
_OMASK = (1 << _OSH) - 1
_WIN = 512
_RING = 8


def _make_gather_kernel(B, D, n_node, num_cores):
    mesh = plsc.VectorSubcoreMesh(core_axis_name="c", subcore_axis_name="s")
    nq = 2 * B
    pad_cols = ((n_node + 127) // 128) * 128
    last_owner = (n_node - 1) >> _OSH
    tail_base = (last_owner << _OSH) + ((pad_cols - (last_owner << _OSH)) // _WIN) * _WIN
    tail_cols = n_node - tail_base
    rows_len = B * D + 16

    @functools.partial(
        pl.kernel,
        out_type=(
            jax.ShapeDtypeStruct((rows_len,), jnp.float32),
            jax.ShapeDtypeStruct((rows_len,), jnp.float32),
        ),
        mesh=mesh,
        compiler_params=pltpu.CompilerParams(
            needs_layout_passes=False, use_tc_tiling_on_sc=True
        ),
        scratch_types=[
            pltpu.VMEM((nq + 16,), jnp.int32),
            pltpu.VMEM((nq + 16,), jnp.int32),
            pltpu.VMEM((nq + 16,), jnp.int32),
            pltpu.VMEM((D, _WIN), jnp.float32),
            pltpu.VMEM((64,), jnp.int32),
            pltpu.VMEM((_RING, 16, 16), jnp.float32),
            pltpu.SemaphoreType.DMA,
        ],
    )
    def gather_kernel(tableT_hbm, ida_hbm, idb_hbm, rowsa_hbm, rowsb_hbm,
                      qv, hits, hits2, win, ends, ring, sem_out):
        wid = lax.axis_index("s") * num_cores + lax.axis_index("c")
        pltpu.sync_copy(ida_hbm, qv.at[pl.ds(0, B)])
        pltpu.sync_copy(idb_hbm, qv.at[pl.ds(B, B)])

        lanes = lax.iota(jnp.int32, 16)
        base_col = wid << _OSH

        def scan_body(i, hp):
            ids = qv[pl.ds(i * 16, 16)]
            own = lax.shift_right_logical(ids, _OSH) == wid
            mc = plsc.cumsum(jnp.where(own, 1, 0))
            packed = lax.shift_left(i * 16 + lanes, _OSH) | (ids & _OMASK)
            plsc.store_scatter(hits, [hp + mc - 1], packed, mask=own)
            return hp + mc[15]

        nhits = lax.fori_loop(0, nq // 16, scan_body, jnp.int32(0))
        nvreg = (nhits + 15) // 16

        def radix_pass(bit, src, dst):
            def make_body(want_one):
                def body(i, p):
                    v = src[pl.ds(i * 16, 16)]
                    b = lax.shift_right_logical(v, bit) & 1
                    m = ((i * 16 + lanes) < nhits) & (b == want_one)
                    mc = plsc.cumsum(jnp.where(m, 1, 0))
                    plsc.store_scatter(dst, [p + mc - 1], v, mask=m)
                    return p + mc[15]

                return body

            p = lax.fori_loop(0, nvreg, make_body(0), jnp.int32(0))
            lax.fori_loop(0, nvreg, make_body(1), p)

        radix_pass(9, hits, hits2)
        radix_pass(10, hits2, hits)
        radix_pass(11, hits, hits2)
        radix_pass(12, hits2, hits)
        radix_pass(13, hits, hits2)
        radix_pass(14, hits2, hits)

        zeros16 = jnp.zeros((16,), jnp.int32)
        for _q in range(4):
            ends[pl.ds(_q * 16, 16)] = zeros16

        def hist_body(i, _):
            v = hits[pl.ds(i * 16, 16)]
            valid = (i * 16 + lanes) < nhits
            w = lax.shift_right_logical(v & _OMASK, 9)
            occ, last = plsc.scan_count(w, valid)
            plsc.addupdate_scatter(ends, [w], occ, mask=last & valid)
            return _

        lax.fori_loop(0, nvreg, hist_body, None)
        carry = jnp.int32(0)
        for _q in range(4):
            cq = plsc.cumsum(ends[pl.ds(_q * 16, 16)]) + carry
            ends[pl.ds(_q * 16, 16)] = cq
            carry = jnp.max(cq)

        def serve_segment(h0, h1, g0):
            def serve_body(gg, g):
                hblk = (h0 // 16 + gg) * 16
                v = hits[pl.ds(pl.multiple_of(hblk, 16), 16)]
                pos = hblk + lanes
                valid = (pos >= h0) & (pos < h1)
                cols = (v & _OMASK) & (_WIN - 1)
                qid = lax.shift_right_logical(v, _OSH)
                slot = lax.rem(g, jnp.int32(_RING))

                @pl.when(g >= _RING)
                def _():
                    for _k in range(16):
                        pltpu.make_async_copy(
                            rowsa_hbm.at[pl.ds(0, 16)], ring.at[0, 0], sem_out
                        ).wait()

                sv = jnp.full((16,), 0, jnp.int32) + slot
                for d in range(D):
                    dv = jnp.full((16,), d, jnp.int32)
                    fv = plsc.load_gather(win, [dv, cols])
                    plsc.store_scatter(ring, [sv, lanes, dv], fv)

                arr_b = qid >= B
                dst_off = jnp.where(
                    valid, jnp.where(arr_b, qid - B, qid) * D, B * D
                )
                ibv = jnp.where(arr_b & valid, 1, 0)
                for j in range(16):
                    @pl.when(ibv[j] == 1)
                    def _():
                        pltpu.async_copy(
                            ring.at[slot, j],
                            rowsb_hbm.at[pl.ds(pl.multiple_of(dst_off[j], 16), D)],
                            sem_out,
                        )

                    @pl.when(ibv[j] == 0)
                    def _():
                        pltpu.async_copy(
                            ring.at[slot, j],
                            rowsa_hbm.at[pl.ds(pl.multiple_of(dst_off[j], 16), D)],
                            sem_out,
                        )

                return g + 1

            ngrp = jnp.where(h1 > h0, (h1 - 1) // 16 - h0 // 16 + 1, 0)
            return lax.fori_loop(0, ngrp, serve_body, g0)

        def end_of(w):
            wv = jnp.full((16,), 0, jnp.int32) + w
            return jnp.max(plsc.load_gather(ends, [wv]))

        def win_body(w, g):
            pltpu.sync_copy(tableT_hbm.at[:, pl.ds(pl.multiple_of(base_col + w * _WIN, 128), _WIN)], win)
            h1 = end_of(w)
            h0 = jnp.where(w > 0, end_of(jnp.maximum(w - 1, 0)), 0)
            return serve_segment(h0, h1, g)

        n_win = jnp.where(
            wid == last_owner,
            (tail_base - (last_owner << _OSH)) // _WIN,
            jnp.where(wid > last_owner, 0, (1 << _OSH) // _WIN),
        )
        g_total = lax.fori_loop(0, n_win, win_body, jnp.int32(0))

        def drain_m(i, _):
            for _k in range(16):
                pltpu.make_async_copy(
                    rowsa_hbm.at[pl.ds(0, 16)], ring.at[0, 0], sem_out
                ).wait()
            return _

        lax.fori_loop(0, jnp.minimum(g_total, _RING), drain_m, None)

    return gather_kernel


def _make_dot_kernel(B, D, b_per_w, tail_base, num_cores):
    mesh = plsc.VectorSubcoreMesh(core_axis_name="c", subcore_axis_name="s")

    @functools.partial(
        pl.kernel,
        out_type=jax.ShapeDtypeStruct((B,), jnp.float32),
        mesh=mesh,
        compiler_params=pltpu.CompilerParams(
            needs_layout_passes=False, use_tc_tiling_on_sc=False
        ),
        scratch_types=[
            pltpu.VMEM((b_per_w * D,), jnp.float32),
            pltpu.VMEM((b_per_w * D,), jnp.float32),
            pltpu.VMEM((b_per_w, D), jnp.float32),
            pltpu.VMEM((b_per_w, D), jnp.float32),
            pltpu.VMEM((b_per_w,), jnp.int32),
            pltpu.VMEM((b_per_w,), jnp.int32),
            pltpu.VMEM((b_per_w,), jnp.int32),
            pltpu.VMEM((b_per_w,), jnp.int32),
            pltpu.VMEM((b_per_w,), jnp.float32),
            pltpu.VMEM((b_per_w,), jnp.float32),
            pltpu.SemaphoreType.DMA,
            pltpu.SemaphoreType.DMA,
            pltpu.SemaphoreType.DMA,
        ],
    )
    def dot_kernel(rowsa_hbm, rowsb_hbm, bias_hbm, ida_hbm, idb_hbm, tail_hbm,
                   out_hbm, ra, rb, ta, tb, idx_a, idx_b, tia, tib,
                   bias_v, out_v, sem, sem_t, sem_b):
        wid = lax.axis_index("s") * num_cores + lax.axis_index("c")
        base = wid * b_per_w
        pltpu.sync_copy(ida_hbm.at[pl.ds(base, b_per_w)], idx_a)
        pltpu.sync_copy(idb_hbm.at[pl.ds(base, b_per_w)], idx_b)
        bias_copy = pltpu.async_copy(bias_hbm.at[idx_b], bias_v, sem_b)

        def clamp_body(blk, _):
            r0 = blk * 16
            va = idx_a[pl.ds(r0, 16)]
            vb = idx_b[pl.ds(r0, 16)]
            tia[pl.ds(r0, 16)] = jnp.maximum(va - tail_base, 0)
            tib[pl.ds(r0, 16)] = jnp.maximum(vb - tail_base, 0)
            return _

        lax.fori_loop(0, b_per_w // 16, clamp_body, None)
        ca = pltpu.async_copy(tail_hbm.at[tia], ta, sem_t)
        cb = pltpu.async_copy(tail_hbm.at[tib], tb, sem_t)
        pltpu.sync_copy(rowsa_hbm.at[pl.ds(base * D, b_per_w * D)], ra)
        pltpu.sync_copy(rowsb_hbm.at[pl.ds(base * D, b_per_w * D)], rb)
        ca.wait()
        cb.wait()
        bias_copy.wait()
        lanes = lax.iota(jnp.int32, 16)

        def body(blk, _):
            row0 = blk * 16
            flat0 = (row0 + lanes) * D
            in_ta = idx_a[pl.ds(row0, 16)] >= tail_base
            in_tb = idx_b[pl.ds(row0, 16)] >= tail_base
            acc = bias_v[pl.ds(row0, 16)]
            for d in range(D):
                va = plsc.load_gather(ra, [flat0 + d])
                vb = plsc.load_gather(rb, [flat0 + d])
                dv = jnp.full((16,), d, jnp.int32)
                va = jnp.where(in_ta, plsc.load_gather(ta, [row0 + lanes, dv]), va)
                vb = jnp.where(in_tb, plsc.load_gather(tb, [row0 + lanes, dv]), vb)
                acc = acc + va * vb
            out_v[pl.ds(row0, 16)] = acc
            return _

        lax.fori_loop(0, b_per_w // 16, body, None)
        pltpu.sync_copy(out_v, out_hbm.at[pl.ds(base, b_per_w)])

    return dot_kernel


def kernel(embedding_matrix, bias, node_id, node_neighbor_id):
    B = node_id.shape[0]
    n_node, D = embedding_matrix.shape
    info = plsc.get_sparse_core_info()
    nw = info.num_cores * info.num_subcores
    b_per_w = B // nw
    ida = node_id.astype(jnp.int32)
    idb = node_neighbor_id.astype(jnp.int32)
    pad_cols = ((n_node + 127) // 128) * 128
    last_owner = (n_node - 1) >> _OSH
    tail_base = (last_owner << _OSH) + (
        (pad_cols - (last_owner << _OSH)) // _WIN
    ) * _WIN
    tail = jax.lax.slice_in_dim(embedding_matrix, tail_base, n_node, axis=0)
    gk = _make_gather_kernel(B, D, n_node, info.num_cores)
    rows_a, rows_b = gk(embedding_matrix.T, ida, idb)
    dk = _make_dot_kernel(B, D, b_per_w, tail_base, info.num_cores)
    return dk(rows_a, rows_b, bias, ida, idb, tail)


# speedup vs baseline: 4.4633x; 1.4369x over previous
import functools

import jax
import jax.numpy as jnp
from jax import lax
from jax.experimental import pallas as pl
from jax.experimental.pallas import tpu as pltpu
from jax.experimental.pallas import tpu_sc as plsc

_OSH = 15
_OMASK = (1 << _OSH) - 1
_WIN = 512
_RING = 8


def _make_gather_kernel(B, D, n_node, num_cores):
    mesh = plsc.VectorSubcoreMesh(core_axis_name="c", subcore_axis_name="s")
    nq = 2 * B
    pad_cols = ((n_node + 127) // 128) * 128
    last_owner = (n_node - 1) >> _OSH
    tail_base = (last_owner << _OSH) + ((pad_cols - (last_owner << _OSH)) // _WIN) * _WIN
    tail_cols = n_node - tail_base
    rows_len = B * D + 16

    @functools.partial(
        pl.kernel,
        out_type=(
            jax.ShapeDtypeStruct((rows_len,), jnp.float32),
            jax.ShapeDtypeStruct((rows_len,), jnp.float32),
        ),
        mesh=mesh,
        compiler_params=pltpu.CompilerParams(
            needs_layout_passes=False, use_tc_tiling_on_sc=True
        ),
        scratch_types=[
            pltpu.VMEM((nq + 16,), jnp.int32),
            pltpu.VMEM((nq + 16,), jnp.int32),
            pltpu.VMEM((nq + 16,), jnp.int32),
            pltpu.VMEM((D, _WIN), jnp.float32),
            pltpu.VMEM((64,), jnp.int32),
            pltpu.VMEM((_RING, 16, 16), jnp.float32),
            pltpu.SemaphoreType.DMA,
        ],
    )
    def gather_kernel(tableT_hbm, ida_hbm, idb_hbm, rowsa_hbm, rowsb_hbm,
                      qv, hits, hits2, win, ends, ring, sem_out):
        wid = lax.axis_index("s") * num_cores + lax.axis_index("c")
        pltpu.sync_copy(ida_hbm, qv.at[pl.ds(0, B)])
        pltpu.sync_copy(idb_hbm, qv.at[pl.ds(B, B)])

        lanes = lax.iota(jnp.int32, 16)
        base_col = wid << _OSH

        def scan_body(i, hp):
            ids = qv[pl.ds(i * 16, 16)]
            own = lax.shift_right_logical(ids, _OSH) == wid
            cnt = plsc.all_reduce_population_count(own)[0]

            @pl.when(cnt > 0)
            def _():
                mc = plsc.cumsum(jnp.where(own, 1, 0))
                packed = lax.shift_left(i * 16 + lanes, _OSH) | (ids & _OMASK)
                plsc.store_scatter(hits, [hp + mc - 1], packed, mask=own)

            return hp + cnt

        nhits = lax.fori_loop(0, nq // 16, scan_body, jnp.int32(0))
        nvreg = (nhits + 15) // 16

        def radix_pass(bit, src, dst):
            def make_body(want_one):
                def body(i, p):
                    v = src[pl.ds(i * 16, 16)]
                    b = lax.shift_right_logical(v, bit) & 1
                    m = ((i * 16 + lanes) < nhits) & (b == want_one)
                    mc = plsc.cumsum(jnp.where(m, 1, 0))
                    plsc.store_scatter(dst, [p + mc - 1], v, mask=m)
                    return p + mc[15]

                return body

            p = lax.fori_loop(0, nvreg, make_body(0), jnp.int32(0))
            lax.fori_loop(0, nvreg, make_body(1), p)

        radix_pass(9, hits, hits2)
        radix_pass(10, hits2, hits)
        radix_pass(11, hits, hits2)
        radix_pass(12, hits2, hits)
        radix_pass(13, hits, hits2)
        radix_pass(14, hits2, hits)

        zeros16 = jnp.zeros((16,), jnp.int32)
        for _q in range(4):
            ends[pl.ds(_q * 16, 16)] = zeros16

        def hist_body(i, _):
            v = hits[pl.ds(i * 16, 16)]
            valid = (i * 16 + lanes) < nhits
            w = lax.shift_right_logical(v & _OMASK, 9)
            occ, last = plsc.scan_count(w, valid)
            plsc.addupdate_scatter(ends, [w], occ, mask=last & valid)
            return _

        lax.fori_loop(0, nvreg, hist_body, None)
        carry = jnp.int32(0)
        for _q in range(4):
            cq = plsc.cumsum(ends[pl.ds(_q * 16, 16)]) + carry
            ends[pl.ds(_q * 16, 16)] = cq
            carry = jnp.max(cq)

        def serve_segment(h0, h1, g0):
            def serve_body(gg, g):
                hblk = (h0 // 16 + gg) * 16
                v = hits[pl.ds(pl.multiple_of(hblk, 16), 16)]
                pos = hblk + lanes
                valid = (pos >= h0) & (pos < h1)
                cols = (v & _OMASK) & (_WIN - 1)
                qid = lax.shift_right_logical(v, _OSH)
                slot = lax.rem(g, jnp.int32(_RING))

                @pl.when(g >= _RING)
                def _():
                    for _k in range(16):
                        pltpu.make_async_copy(
                            rowsa_hbm.at[pl.ds(0, 16)], ring.at[0, 0], sem_out
                        ).wait()

                sv = jnp.full((16,), 0, jnp.int32) + slot
                for d in range(D):
                    dv = jnp.full((16,), d, jnp.int32)
                    fv = plsc.load_gather(win, [dv, cols])
                    plsc.store_scatter(ring, [sv, lanes, dv], fv)

                arr_b = qid >= B
                dst_off = jnp.where(
                    valid, jnp.where(arr_b, qid - B, qid) * D, B * D
                )
                ibv = jnp.where(arr_b & valid, 1, 0)
                for j in range(16):
                    @pl.when(ibv[j] == 1)
                    def _():
                        pltpu.async_copy(
                            ring.at[slot, j],
                            rowsb_hbm.at[pl.ds(pl.multiple_of(dst_off[j], 16), D)],
                            sem_out,
                        )

                    @pl.when(ibv[j] == 0)
                    def _():
                        pltpu.async_copy(
                            ring.at[slot, j],
                            rowsa_hbm.at[pl.ds(pl.multiple_of(dst_off[j], 16), D)],
                            sem_out,
                        )

                return g + 1

            ngrp = jnp.where(h1 > h0, (h1 - 1) // 16 - h0 // 16 + 1, 0)
            return lax.fori_loop(0, ngrp, serve_body, g0)

        def end_of(w):
            wv = jnp.full((16,), 0, jnp.int32) + w
            return jnp.max(plsc.load_gather(ends, [wv]))

        def win_body(w, g):
            pltpu.sync_copy(tableT_hbm.at[:, pl.ds(pl.multiple_of(base_col + w * _WIN, 128), _WIN)], win)
            h1 = end_of(w)
            h0 = jnp.where(w > 0, end_of(jnp.maximum(w - 1, 0)), 0)
            return serve_segment(h0, h1, g)

        n_win = jnp.where(
            wid == last_owner,
            (tail_base - (last_owner << _OSH)) // _WIN,
            jnp.where(wid > last_owner, 0, (1 << _OSH) // _WIN),
        )
        g_total = lax.fori_loop(0, n_win, win_body, jnp.int32(0))

        def drain_m(i, _):
            for _k in range(16):
                pltpu.make_async_copy(
                    rowsa_hbm.at[pl.ds(0, 16)], ring.at[0, 0], sem_out
                ).wait()
            return _

        lax.fori_loop(0, jnp.minimum(g_total, _RING), drain_m, None)

    return gather_kernel


def _make_dot_kernel(B, D, b_per_w, tail_base, num_cores):
    mesh = plsc.VectorSubcoreMesh(core_axis_name="c", subcore_axis_name="s")

    @functools.partial(
        pl.kernel,
        out_type=jax.ShapeDtypeStruct((B,), jnp.float32),
        mesh=mesh,
        compiler_params=pltpu.CompilerParams(
            needs_layout_passes=False, use_tc_tiling_on_sc=False
        ),
        scratch_types=[
            pltpu.VMEM((b_per_w * D,), jnp.float32),
            pltpu.VMEM((b_per_w * D,), jnp.float32),
            pltpu.VMEM((b_per_w, D), jnp.float32),
            pltpu.VMEM((b_per_w, D), jnp.float32),
            pltpu.VMEM((b_per_w,), jnp.int32),
            pltpu.VMEM((b_per_w,), jnp.int32),
            pltpu.VMEM((b_per_w,), jnp.int32),
            pltpu.VMEM((b_per_w,), jnp.int32),
            pltpu.VMEM((b_per_w,), jnp.float32),
            pltpu.VMEM((b_per_w,), jnp.float32),
            pltpu.SemaphoreType.DMA,
            pltpu.SemaphoreType.DMA,
            pltpu.SemaphoreType.DMA,
        ],
    )
    def dot_kernel(rowsa_hbm, rowsb_hbm, bias_hbm, ida_hbm, idb_hbm, tail_hbm,
                   out_hbm, ra, rb, ta, tb, idx_a, idx_b, tia, tib,
                   bias_v, out_v, sem, sem_t, sem_b):
        wid = lax.axis_index("s") * num_cores + lax.axis_index("c")
        base = wid * b_per_w
        pltpu.sync_copy(ida_hbm.at[pl.ds(base, b_per_w)], idx_a)
        pltpu.sync_copy(idb_hbm.at[pl.ds(base, b_per_w)], idx_b)
        bias_copy = pltpu.async_copy(bias_hbm.at[idx_b], bias_v, sem_b)

        lanes0 = lax.iota(jnp.int32, 16)

        def clamp_body(blk, mx):
            r0 = blk * 16
            va = idx_a[pl.ds(r0, 16)]
            vb = idx_b[pl.ds(r0, 16)]
            spread = (r0 + lanes0) & 63
            tia[pl.ds(r0, 16)] = jnp.where(va >= tail_base, va - tail_base, spread)
            tib[pl.ds(r0, 16)] = jnp.where(vb >= tail_base, vb - tail_base, spread)
            return jnp.maximum(mx, jnp.maximum(jnp.max(va), jnp.max(vb)))

        mx = lax.fori_loop(0, b_per_w // 16, clamp_body, jnp.int32(0))
        has_tail = mx >= tail_base

        @pl.when(has_tail)
        def _():
            pltpu.async_copy(tail_hbm.at[tia], ta, sem_t).wait()
            pltpu.async_copy(tail_hbm.at[tib], tb, sem_t).wait()
        pltpu.sync_copy(rowsa_hbm.at[pl.ds(base * D, b_per_w * D)], ra)
        pltpu.sync_copy(rowsb_hbm.at[pl.ds(base * D, b_per_w * D)], rb)
        bias_copy.wait()
        lanes = lax.iota(jnp.int32, 16)

        def body(blk, _):
            row0 = blk * 16
            flat0 = (row0 + lanes) * D
            in_ta = idx_a[pl.ds(row0, 16)] >= tail_base
            in_tb = idx_b[pl.ds(row0, 16)] >= tail_base
            acc = bias_v[pl.ds(row0, 16)]
            for d in range(D):
                va = plsc.load_gather(ra, [flat0 + d])
                vb = plsc.load_gather(rb, [flat0 + d])
                dv = jnp.full((16,), d, jnp.int32)
                va = jnp.where(in_ta, plsc.load_gather(ta, [row0 + lanes, dv]), va)
                vb = jnp.where(in_tb, plsc.load_gather(tb, [row0 + lanes, dv]), vb)
                acc = acc + va * vb
            out_v[pl.ds(row0, 16)] = acc
            return _

        lax.fori_loop(0, b_per_w // 16, body, None)
        pltpu.sync_copy(out_v, out_hbm.at[pl.ds(base, b_per_w)])

    return dot_kernel


def kernel(embedding_matrix, bias, node_id, node_neighbor_id):
    B = node_id.shape[0]
    n_node, D = embedding_matrix.shape
    info = plsc.get_sparse_core_info()
    nw = info.num_cores * info.num_subcores
    b_per_w = B // nw
    ida = node_id.astype(jnp.int32)
    idb = node_neighbor_id.astype(jnp.int32)
    pad_cols = ((n_node + 127) // 128) * 128
    last_owner = (n_node - 1) >> _OSH
    tail_base = (last_owner << _OSH) + (
        (pad_cols - (last_owner << _OSH)) // _WIN
    ) * _WIN
    tail = jax.lax.slice_in_dim(embedding_matrix, tail_base, n_node, axis=0)
    gk = _make_gather_kernel(B, D, n_node, info.num_cores)
    rows_a, rows_b = gk(embedding_matrix.T, ida, idb)
    dk = _make_dot_kernel(B, D, b_per_w, tail_base, info.num_cores)
    return dk(rows_a, rows_b, bias, ida, idb, tail)


# speedup vs baseline: 4.4868x; 1.0053x over previous
import functools

import jax
import jax.numpy as jnp
from jax import lax
from jax.experimental import pallas as pl
from jax.experimental.pallas import tpu as pltpu
from jax.experimental.pallas import tpu_sc as plsc

_OSH = 15
_OMASK = (1 << _OSH) - 1
_WIN = 512
_RING = 8


def _make_gather_kernel(B, D, n_node, num_cores):
    mesh = plsc.VectorSubcoreMesh(core_axis_name="c", subcore_axis_name="s")
    nq = 2 * B
    pad_cols = ((n_node + 127) // 128) * 128
    last_owner = (n_node - 1) >> _OSH
    tail_base = (last_owner << _OSH) + ((pad_cols - (last_owner << _OSH)) // _WIN) * _WIN
    tail_cols = n_node - tail_base
    rows_len = B * D + 16

    @functools.partial(
        pl.kernel,
        out_type=(
            jax.ShapeDtypeStruct((rows_len,), jnp.float32),
            jax.ShapeDtypeStruct((rows_len,), jnp.float32),
        ),
        mesh=mesh,
        compiler_params=pltpu.CompilerParams(
            needs_layout_passes=False, use_tc_tiling_on_sc=True
        ),
        scratch_types=[
            pltpu.VMEM((nq + 16,), jnp.int32),
            pltpu.VMEM((nq + 16,), jnp.int32),
            pltpu.VMEM((nq + 16,), jnp.int32),
            pltpu.VMEM((D, _WIN), jnp.float32),
            pltpu.VMEM((64,), jnp.int32),
            pltpu.VMEM((_RING, 16, 16), jnp.float32),
            pltpu.SemaphoreType.DMA,
        ],
    )
    def gather_kernel(tableT_hbm, ida_hbm, idb_hbm, rowsa_hbm, rowsb_hbm,
                      qv, hits, hits2, win, ends, ring, sem_out):
        wid = lax.axis_index("s") * num_cores + lax.axis_index("c")
        pltpu.sync_copy(ida_hbm, qv.at[pl.ds(0, B)])
        pltpu.sync_copy(idb_hbm, qv.at[pl.ds(B, B)])

        lanes = lax.iota(jnp.int32, 16)
        base_col = wid << _OSH

        def scan_body(i4, hp):
            for u in range(4):
                i = i4 * 4 + u
                ids = qv[pl.ds(i * 16, 16)]
                own = lax.shift_right_logical(ids, _OSH) == wid
                cnt = plsc.all_reduce_population_count(own)[0]

                @pl.when(cnt > 0)
                def _(i=i, ids=ids, own=own, hp=hp):
                    mc = plsc.cumsum(jnp.where(own, 1, 0))
                    packed = lax.shift_left(i * 16 + lanes, _OSH) | (ids & _OMASK)
                    plsc.store_scatter(hits, [hp + mc - 1], packed, mask=own)

                hp = hp + cnt
            return hp

        nhits = lax.fori_loop(0, nq // 64, scan_body, jnp.int32(0))
        nvreg = (nhits + 15) // 16

        def radix_pass(bit, src, dst):
            def make_body(want_one):
                def body(i, p):
                    v = src[pl.ds(i * 16, 16)]
                    b = lax.shift_right_logical(v, bit) & 1
                    m = ((i * 16 + lanes) < nhits) & (b == want_one)
                    mc = plsc.cumsum(jnp.where(m, 1, 0))
                    plsc.store_scatter(dst, [p + mc - 1], v, mask=m)
                    return p + mc[15]

                return body

            p = lax.fori_loop(0, nvreg, make_body(0), jnp.int32(0))
            lax.fori_loop(0, nvreg, make_body(1), p)

        radix_pass(9, hits, hits2)
        radix_pass(10, hits2, hits)
        radix_pass(11, hits, hits2)
        radix_pass(12, hits2, hits)
        radix_pass(13, hits, hits2)
        radix_pass(14, hits2, hits)

        zeros16 = jnp.zeros((16,), jnp.int32)
        for _q in range(4):
            ends[pl.ds(_q * 16, 16)] = zeros16

        def hist_body(i, _):
            v = hits[pl.ds(i * 16, 16)]
            valid = (i * 16 + lanes) < nhits
            w = lax.shift_right_logical(v & _OMASK, 9)
            occ, last = plsc.scan_count(w, valid)
            plsc.addupdate_scatter(ends, [w], occ, mask=last & valid)
            return _

        lax.fori_loop(0, nvreg, hist_body, None)
        carry = jnp.int32(0)
        for _q in range(4):
            cq = plsc.cumsum(ends[pl.ds(_q * 16, 16)]) + carry
            ends[pl.ds(_q * 16, 16)] = cq
            carry = jnp.max(cq)

        def serve_segment(h0, h1, g0):
            def serve_body(gg, g):
                hblk = (h0 // 16 + gg) * 16
                v = hits[pl.ds(pl.multiple_of(hblk, 16), 16)]
                pos = hblk + lanes
                valid = (pos >= h0) & (pos < h1)
                cols = (v & _OMASK) & (_WIN - 1)
                qid = lax.shift_right_logical(v, _OSH)
                slot = lax.rem(g, jnp.int32(_RING))

                @pl.when(g >= _RING)
                def _():
                    for _k in range(16):
                        pltpu.make_async_copy(
                            rowsa_hbm.at[pl.ds(0, 16)], ring.at[0, 0], sem_out
                        ).wait()

                sv = jnp.full((16,), 0, jnp.int32) + slot
                for d in range(D):
                    dv = jnp.full((16,), d, jnp.int32)
                    fv = plsc.load_gather(win, [dv, cols])
                    plsc.store_scatter(ring, [sv, lanes, dv], fv)

                arr_b = qid >= B
                dst_off = jnp.where(
                    valid, jnp.where(arr_b, qid - B, qid) * D, B * D
                )
                ibv = jnp.where(arr_b & valid, 1, 0)
                for j in range(16):
                    @pl.when(ibv[j] == 1)
                    def _():
                        pltpu.async_copy(
                            ring.at[slot, j],
                            rowsb_hbm.at[pl.ds(pl.multiple_of(dst_off[j], 16), D)],
                            sem_out,
                        )

                    @pl.when(ibv[j] == 0)
                    def _():
                        pltpu.async_copy(
                            ring.at[slot, j],
                            rowsa_hbm.at[pl.ds(pl.multiple_of(dst_off[j], 16), D)],
                            sem_out,
                        )

                return g + 1

            ngrp = jnp.where(h1 > h0, (h1 - 1) // 16 - h0 // 16 + 1, 0)
            return lax.fori_loop(0, ngrp, serve_body, g0)

        def end_of(w):
            wv = jnp.full((16,), 0, jnp.int32) + w
            return jnp.max(plsc.load_gather(ends, [wv]))

        def win_body(w, g):
            pltpu.sync_copy(tableT_hbm.at[:, pl.ds(pl.multiple_of(base_col + w * _WIN, 128), _WIN)], win)
            h1 = end_of(w)
            h0 = jnp.where(w > 0, end_of(jnp.maximum(w - 1, 0)), 0)
            return serve_segment(h0, h1, g)

        n_win = jnp.where(
            wid == last_owner,
            (tail_base - (last_owner << _OSH)) // _WIN,
            jnp.where(wid > last_owner, 0, (1 << _OSH) // _WIN),
        )
        g_total = lax.fori_loop(0, n_win, win_body, jnp.int32(0))

        def drain_m(i, _):
            for _k in range(16):
                pltpu.make_async_copy(
                    rowsa_hbm.at[pl.ds(0, 16)], ring.at[0, 0], sem_out
                ).wait()
            return _

        lax.fori_loop(0, jnp.minimum(g_total, _RING), drain_m, None)

    return gather_kernel


def _make_dot_kernel(B, D, b_per_w, tail_base, num_cores):
    mesh = plsc.VectorSubcoreMesh(core_axis_name="c", subcore_axis_name="s")

    @functools.partial(
        pl.kernel,
        out_type=jax.ShapeDtypeStruct((B,), jnp.float32),
        mesh=mesh,
        compiler_params=pltpu.CompilerParams(
            needs_layout_passes=False, use_tc_tiling_on_sc=False
        ),
        scratch_types=[
            pltpu.VMEM((b_per_w * D,), jnp.float32),
            pltpu.VMEM((b_per_w * D,), jnp.float32),
            pltpu.VMEM((b_per_w, D), jnp.float32),
            pltpu.VMEM((b_per_w, D), jnp.float32),
            pltpu.VMEM((b_per_w,), jnp.int32),
            pltpu.VMEM((b_per_w,), jnp.int32),
            pltpu.VMEM((b_per_w,), jnp.int32),
            pltpu.VMEM((b_per_w,), jnp.int32),
            pltpu.VMEM((b_per_w,), jnp.float32),
            pltpu.VMEM((b_per_w,), jnp.float32),
            pltpu.SemaphoreType.DMA,
            pltpu.SemaphoreType.DMA,
            pltpu.SemaphoreType.DMA,
        ],
    )
    def dot_kernel(rowsa_hbm, rowsb_hbm, bias_hbm, ida_hbm, idb_hbm, tail_hbm,
                   out_hbm, ra, rb, ta, tb, idx_a, idx_b, tia, tib,
                   bias_v, out_v, sem, sem_t, sem_b):
        wid = lax.axis_index("s") * num_cores + lax.axis_index("c")
        base = wid * b_per_w
        pltpu.sync_copy(ida_hbm.at[pl.ds(base, b_per_w)], idx_a)
        pltpu.sync_copy(idb_hbm.at[pl.ds(base, b_per_w)], idx_b)
        bias_copy = pltpu.async_copy(bias_hbm.at[idx_b], bias_v, sem_b)

        lanes0 = lax.iota(jnp.int32, 16)

        def clamp_body(blk, mx):
            r0 = blk * 16
            va = idx_a[pl.ds(r0, 16)]
            vb = idx_b[pl.ds(r0, 16)]
            spread = (r0 + lanes0) & 63
            tia[pl.ds(r0, 16)] = jnp.where(va >= tail_base, va - tail_base, spread)
            tib[pl.ds(r0, 16)] = jnp.where(vb >= tail_base, vb - tail_base, spread)
            return jnp.maximum(mx, jnp.maximum(jnp.max(va), jnp.max(vb)))

        mx = lax.fori_loop(0, b_per_w // 16, clamp_body, jnp.int32(0))
        has_tail = mx >= tail_base

        @pl.when(has_tail)
        def _():
            pltpu.async_copy(tail_hbm.at[tia], ta, sem_t).wait()
            pltpu.async_copy(tail_hbm.at[tib], tb, sem_t).wait()
        pltpu.sync_copy(rowsa_hbm.at[pl.ds(base * D, b_per_w * D)], ra)
        pltpu.sync_copy(rowsb_hbm.at[pl.ds(base * D, b_per_w * D)], rb)
        bias_copy.wait()
        lanes = lax.iota(jnp.int32, 16)

        def body(blk, _):
            row0 = blk * 16
            flat0 = (row0 + lanes) * D
            in_ta = idx_a[pl.ds(row0, 16)] >= tail_base
            in_tb = idx_b[pl.ds(row0, 16)] >= tail_base
            acc = bias_v[pl.ds(row0, 16)]
            for d in range(D):
                va = plsc.load_gather(ra, [flat0 + d])
                vb = plsc.load_gather(rb, [flat0 + d])
                dv = jnp.full((16,), d, jnp.int32)
                va = jnp.where(in_ta, plsc.load_gather(ta, [row0 + lanes, dv]), va)
                vb = jnp.where(in_tb, plsc.load_gather(tb, [row0 + lanes, dv]), vb)
                acc = acc + va * vb
            out_v[pl.ds(row0, 16)] = acc
            return _

        lax.fori_loop(0, b_per_w // 16, body, None)
        pltpu.sync_copy(out_v, out_hbm.at[pl.ds(base, b_per_w)])

    return dot_kernel


def kernel(embedding_matrix, bias, node_id, node_neighbor_id):
    B = node_id.shape[0]
    n_node, D = embedding_matrix.shape
    info = plsc.get_sparse_core_info()
    nw = info.num_cores * info.num_subcores
    b_per_w = B // nw
    ida = node_id.astype(jnp.int32)
    idb = node_neighbor_id.astype(jnp.int32)
    pad_cols = ((n_node + 127) // 128) * 128
    last_owner = (n_node - 1) >> _OSH
    tail_base = (last_owner << _OSH) + (
        (pad_cols - (last_owner << _OSH)) // _WIN
    ) * _WIN
    tail = jax.lax.slice_in_dim(embedding_matrix, tail_base, n_node, axis=0)
    gk = _make_gather_kernel(B, D, n_node, info.num_cores)
    rows_a, rows_b = gk(embedding_matrix.T, ida, idb)
    dk = _make_dot_kernel(B, D, b_per_w, tail_base, info.num_cores)
    return dk(rows_a, rows_b, bias, ida, idb, tail)
